# baseline (speedup 1.0000x reference)
import functools

import jax
import jax.numpy as jnp
from jax import lax
from jax.experimental import pallas as pl
from jax.experimental.pallas import tpu as pltpu
from jax.experimental.pallas import tpu_sc as plsc

NUM_EMB = 1_000_000
DIM = 64
BATCH = 4096
SEQ = 200

_INFO = plsc.get_sparse_core_info()
NC = _INFO.num_cores
NS = _INFO.num_subcores
NW = NC * NS
ROWS_PER_W = BATCH // NW
NBUF = 4
NIT = ROWS_PER_W // NBUF


def _emb_body(table_hbm, idx_hbm, out_hbm, idx_v, rows_v, *sems):
    gsems = sems[:NBUF]
    wsems = sems[NBUF:]
    c = lax.axis_index("c")
    s = lax.axis_index("s")
    wid = s * NC + c
    row0 = wid * ROWS_PER_W
    pltpu.sync_copy(idx_hbm.at[pl.ds(row0, ROWS_PER_W)], idx_v)

    def gather(j, b):
        pltpu.make_async_copy(
            table_hbm.at[idx_v.at[j]],
            rows_v.at[b],
            gsems[b],
        ).start()

    def drain_gather(j, b):
        pltpu.make_async_copy(
            table_hbm.at[idx_v.at[j]],
            rows_v.at[b],
            gsems[b],
        ).wait()

    def writeback(j, b):
        pltpu.make_async_copy(
            rows_v.at[b],
            out_hbm.at[row0 + j],
            wsems[b],
        ).start()

    def drain_writeback(j, b):
        pltpu.make_async_copy(
            rows_v.at[b],
            out_hbm.at[row0 + j],
            wsems[b],
        ).wait()

    for b in range(NBUF - 1):
        gather(b, b)

    def step(it, carry):
        j0 = it * NBUF
        for b in range(NBUF):
            j = j0 + b
            drain_gather(j, b)
            writeback(j, b)
            bn = (b + NBUF - 1) % NBUF
            jn = j + NBUF - 1

            if b == 0:
                @pl.when(jn < ROWS_PER_W)
                def _():
                    @pl.when(it > 0)
                    def _():
                        drain_writeback(j - 1, bn)

                    gather(jn, bn)
            else:
                @pl.when(jn < ROWS_PER_W)
                def _():
                    drain_writeback(j - 1, bn)
                    gather(jn, bn)

        return carry

    lax.fori_loop(0, NIT, step, 0)
    for j in range(ROWS_PER_W - NBUF + 1, ROWS_PER_W):
        drain_writeback(j, j % NBUF)
    drain_writeback(ROWS_PER_W - NBUF, (ROWS_PER_W - NBUF) % NBUF)


_emb_call = functools.partial(
    pl.kernel,
    out_type=jax.ShapeDtypeStruct((BATCH, SEQ, DIM), jnp.float32),
    mesh=plsc.VectorSubcoreMesh(core_axis_name="c", subcore_axis_name="s"),
    scratch_types=[
        pltpu.VMEM((ROWS_PER_W, SEQ), jnp.int32),
        pltpu.VMEM((NBUF, SEQ, DIM), jnp.float32),
    ]
    + [pltpu.SemaphoreType.DMA] * (2 * NBUF),
    compiler_params=pltpu.CompilerParams(use_tc_tiling_on_sc=False),
)(_emb_body)


@jax.jit
def kernel(x, embed_mat):
    return _emb_call(embed_mat, x)
